# Initial kernel scaffold; baseline (speedup 1.0000x reference)
import jax
import jax.numpy as jnp
from jax.experimental import pallas as pl


def kernel(X, row_missing_idx):
    raise NotImplementedError("write your pallas kernel here")


# speedup vs baseline: 7.2342x; 7.2342x over previous
import functools

import jax
import jax.numpy as jnp
from jax import lax
from jax.experimental import pallas as pl
from jax.experimental.pallas import tpu as pltpu
from jax.experimental.pallas import tpu_sc as plsc

N = 1_000_000
M = 500_000
NW = 32
CHUNK = 31_248
LAST_CHUNK = N - (NW - 1) * CHUNK
BUF = LAST_CHUNK
W = 10_000
NWIN = M // W
L = 16


def _make_sc_kernel():
    mesh = plsc.VectorSubcoreMesh(core_axis_name="c", subcore_axis_name="s",
                                  num_cores=2, num_subcores=16)

    @functools.partial(
        pl.kernel,
        out_type=jax.ShapeDtypeStruct((N,), jnp.int32),
        mesh=mesh,
        scratch_types=[
            pltpu.VMEM((W,), jnp.int32),
            pltpu.VMEM((BUF,), jnp.int32),
        ],
        compiler_params=pltpu.CompilerParams(needs_layout_passes=False),
    )
    def scatter_kernel(idx_hbm, out_hbm, win_v, out_v):
        wid = lax.axis_index("s") * 2 + lax.axis_index("c")
        base = wid * CHUNK
        rng = jnp.where(wid == NW - 1, LAST_CHUNK, CHUNK).astype(jnp.uint32)
        lane = lax.iota(jnp.int32, L)
        zeros = jnp.zeros((L,), jnp.int32)

        @pl.loop(0, BUF // L)
        def _zero(i):
            out_v[pl.ds(i * L, L)] = zeros

        @pl.loop(0, NWIN)
        def _win(w):
            pltpu.sync_copy(idx_hbm.at[pl.ds(w * W, W)], win_v)
            jbase = w * W

            @pl.loop(0, W // L, unroll=8)
            def _vec(i):
                vidx = win_v[pl.ds(i * L, L)]
                loc = vidx - base
                mask = loc.astype(jnp.uint32) < rng
                jv = (jbase + i * L) + lane
                plsc.store_scatter(out_v, [loc], jv, mask=mask)

        @pl.when(wid == NW - 1)
        def _store_last():
            pltpu.sync_copy(out_v, out_hbm.at[pl.ds(base, LAST_CHUNK)])

        @pl.when(wid != NW - 1)
        def _store():
            pltpu.sync_copy(out_v.at[pl.ds(0, CHUNK)],
                            out_hbm.at[pl.ds(base, CHUNK)])

    return scatter_kernel


_sc_kernel = _make_sc_kernel()


def kernel(X, row_missing_idx):
    del X
    return _sc_kernel(row_missing_idx)


# speedup vs baseline: 21.1312x; 2.9210x over previous
import functools

import jax
import jax.numpy as jnp
from jax import lax
from jax.experimental import pallas as pl
from jax.experimental.pallas import tpu as pltpu
from jax.experimental.pallas import tpu_sc as plsc

N = 1_000_000
M = 500_000
NW = 32
CHUNK = 31_248
LAST_CHUNK = N - (NW - 1) * CHUNK
BUF = LAST_CHUNK
W = 10_000
NWIN = M // W
L = 16


def _make_sc_kernel():
    mesh = plsc.VectorSubcoreMesh(core_axis_name="c", subcore_axis_name="s",
                                  num_cores=2, num_subcores=16)

    @functools.partial(
        pl.kernel,
        out_type=jax.ShapeDtypeStruct((N,), jnp.int32),
        mesh=mesh,
        scratch_types=[
            pltpu.VMEM((W,), jnp.int32),
            pltpu.VMEM((BUF,), jnp.int32),
        ],
        compiler_params=pltpu.CompilerParams(needs_layout_passes=False),
    )
    def scatter_kernel(idx_hbm, out_hbm, win_v, out_v):
        wid = lax.axis_index("s") * 2 + lax.axis_index("c")
        base = wid * CHUNK
        rng = jnp.where(wid == NW - 1, LAST_CHUNK, CHUNK).astype(jnp.uint32)
        lane = lax.iota(jnp.int32, L)
        zeros = jnp.zeros((L,), jnp.int32)

        @pl.loop(0, BUF // L)
        def _zero(i):
            out_v[pl.ds(i * L, L)] = zeros

        @pl.loop(0, NWIN)
        def _win(w):
            pltpu.sync_copy(idx_hbm.at[pl.ds(w * W, W)], win_v)
            jbase = w * W

            @plsc.parallel_loop(0, W // L, unroll=8)
            def _vec(i):
                vidx = win_v[pl.ds(i * L, L)]
                loc = vidx - base
                mask = loc.astype(jnp.uint32) < rng
                jv = (jbase + i * L) + lane
                plsc.store_scatter(out_v, [loc], jv, mask=mask)

        @pl.when(wid == NW - 1)
        def _store_last():
            pltpu.sync_copy(out_v, out_hbm.at[pl.ds(base, LAST_CHUNK)])

        @pl.when(wid != NW - 1)
        def _store():
            pltpu.sync_copy(out_v.at[pl.ds(0, CHUNK)],
                            out_hbm.at[pl.ds(base, CHUNK)])

    return scatter_kernel


_sc_kernel = _make_sc_kernel()


def kernel(X, row_missing_idx):
    del X
    return _sc_kernel(row_missing_idx)


# speedup vs baseline: 28.4901x; 1.3482x over previous
import functools

import jax
import jax.numpy as jnp
from jax import lax
from jax.experimental import pallas as pl
from jax.experimental.pallas import tpu as pltpu
from jax.experimental.pallas import tpu_sc as plsc

N = 1_000_000
M = 500_000
NW = 32
CHUNK = 31_248
LAST_CHUNK = N - (NW - 1) * CHUNK
BUF = LAST_CHUNK
W = 10_000
NWIN = M // W
L = 16


def _make_sc_kernel():
    mesh = plsc.VectorSubcoreMesh(core_axis_name="c", subcore_axis_name="s",
                                  num_cores=2, num_subcores=16)

    @functools.partial(
        pl.kernel,
        out_type=jax.ShapeDtypeStruct((N,), jnp.int32),
        mesh=mesh,
        scratch_types=[
            pltpu.VMEM((W,), jnp.int32),
            pltpu.VMEM((W,), jnp.int32),
            pltpu.VMEM((BUF,), jnp.int32),
            pltpu.SemaphoreType.DMA,
            pltpu.SemaphoreType.DMA,
        ],
        compiler_params=pltpu.CompilerParams(needs_layout_passes=False),
    )
    def scatter_kernel(idx_hbm, out_hbm, win0_v, win1_v, out_v, sem0, sem1):
        wid = lax.axis_index("s") * 2 + lax.axis_index("c")
        base = wid * CHUNK
        rng = jnp.where(wid == NW - 1, LAST_CHUNK, CHUNK).astype(jnp.uint32)
        lane = lax.iota(jnp.int32, L)
        zeros = jnp.zeros((L,), jnp.int32)

        def wait(win_v, sem):
            pltpu.make_async_copy(idx_hbm.at[pl.ds(0, W)], win_v, sem).wait()

        def process(win_v, jbase):
            @plsc.parallel_loop(0, W // L, unroll=8)
            def _vec(i):
                vidx = win_v[pl.ds(i * L, L)]
                loc = vidx - base
                mask = loc.astype(jnp.uint32) < rng
                jv = (jbase + i * L) + lane
                plsc.store_scatter(out_v, [loc], jv, mask=mask)

        pltpu.async_copy(idx_hbm.at[pl.ds(0, W)], win0_v, sem0)

        @pl.loop(0, BUF // L)
        def _zero(i):
            out_v[pl.ds(i * L, L)] = zeros

        @pl.loop(0, NWIN // 2)
        def _win(t):
            w0 = 2 * t
            pltpu.async_copy(idx_hbm.at[pl.ds((w0 + 1) * W, W)], win1_v, sem1)
            wait(win0_v, sem0)
            process(win0_v, w0 * W)

            @pl.when(w0 + 2 < NWIN)
            def _prefetch():
                pltpu.async_copy(idx_hbm.at[pl.ds((w0 + 2) * W, W)], win0_v, sem0)

            wait(win1_v, sem1)
            process(win1_v, (w0 + 1) * W)

        @pl.when(wid == NW - 1)
        def _store_last():
            pltpu.sync_copy(out_v, out_hbm.at[pl.ds(base, LAST_CHUNK)])

        @pl.when(wid != NW - 1)
        def _store():
            pltpu.sync_copy(out_v.at[pl.ds(0, CHUNK)],
                            out_hbm.at[pl.ds(base, CHUNK)])

    return scatter_kernel


_sc_kernel = _make_sc_kernel()


def kernel(X, row_missing_idx):
    del X
    return _sc_kernel(row_missing_idx)


# speedup vs baseline: 31.5322x; 1.1068x over previous
import functools

import jax
import jax.numpy as jnp
from jax import lax
from jax.experimental import pallas as pl
from jax.experimental.pallas import tpu as pltpu
from jax.experimental.pallas import tpu_sc as plsc

N = 1_000_000
M = 500_000
NW = 32
CHUNK = 31_248
LAST_CHUNK = N - (NW - 1) * CHUNK
BUF = LAST_CHUNK
W = 10_000
NWIN = M // W
L = 16


def _make_sc_kernel():
    mesh = plsc.VectorSubcoreMesh(core_axis_name="c", subcore_axis_name="s",
                                  num_cores=2, num_subcores=16)

    @functools.partial(
        pl.kernel,
        out_type=jax.ShapeDtypeStruct((N,), jnp.int32),
        mesh=mesh,
        scratch_types=[
            pltpu.VMEM((W,), jnp.int32),
            pltpu.VMEM((W,), jnp.int32),
            pltpu.VMEM((BUF,), jnp.int32),
            pltpu.SemaphoreType.DMA,
            pltpu.SemaphoreType.DMA,
        ],
        compiler_params=pltpu.CompilerParams(needs_layout_passes=False),
    )
    def scatter_kernel(idx_hbm, out_hbm, win0_v, win1_v, out_v, sem0, sem1):
        wid = lax.axis_index("s") * 2 + lax.axis_index("c")
        base = wid * CHUNK
        rng = jnp.where(wid == NW - 1, LAST_CHUNK, CHUNK).astype(jnp.uint32)
        lane = lax.iota(jnp.int32, L)
        zeros = jnp.zeros((L,), jnp.int32)

        def wait(win_v, sem):
            pltpu.make_async_copy(idx_hbm.at[pl.ds(0, W)], win_v, sem).wait()

        def process(win_v, jbase):
            @plsc.parallel_loop(0, W // L, unroll=16)
            def _vec(i):
                vidx = win_v[pl.ds(i * L, L)]
                loc = vidx - base
                mask = loc.astype(jnp.uint32) < rng
                jv = (jbase + i * L) + lane
                plsc.store_scatter(out_v, [loc], jv, mask=mask)

        pltpu.async_copy(idx_hbm.at[pl.ds(0, W)], win0_v, sem0)

        @plsc.parallel_loop(0, BUF // L, unroll=8)
        def _zero(i):
            out_v[pl.ds(i * L, L)] = zeros

        @pl.loop(0, NWIN // 2)
        def _win(t):
            w0 = 2 * t
            pltpu.async_copy(idx_hbm.at[pl.ds((w0 + 1) * W, W)], win1_v, sem1)
            wait(win0_v, sem0)
            process(win0_v, w0 * W)

            @pl.when(w0 + 2 < NWIN)
            def _prefetch():
                pltpu.async_copy(idx_hbm.at[pl.ds((w0 + 2) * W, W)], win0_v, sem0)

            wait(win1_v, sem1)
            process(win1_v, (w0 + 1) * W)

        @pl.when(wid == NW - 1)
        def _store_last():
            pltpu.sync_copy(out_v, out_hbm.at[pl.ds(base, LAST_CHUNK)])

        @pl.when(wid != NW - 1)
        def _store():
            pltpu.sync_copy(out_v.at[pl.ds(0, CHUNK)],
                            out_hbm.at[pl.ds(base, CHUNK)])

    return scatter_kernel


_sc_kernel = _make_sc_kernel()


def kernel(X, row_missing_idx):
    del X
    return _sc_kernel(row_missing_idx)


# speedup vs baseline: 33.2806x; 1.0554x over previous
import functools

import jax
import jax.numpy as jnp
from jax import lax
from jax.experimental import pallas as pl
from jax.experimental.pallas import tpu as pltpu
from jax.experimental.pallas import tpu_sc as plsc

N = 1_000_000
M = 500_000
NW = 32
CHUNK = 31_248
LAST_CHUNK = N - (NW - 1) * CHUNK
BUF = LAST_CHUNK
W = 20_000
NWIN = M // W
NPAIR = NWIN // 2
L = 16


def _make_sc_kernel():
    mesh = plsc.VectorSubcoreMesh(core_axis_name="c", subcore_axis_name="s",
                                  num_cores=2, num_subcores=16)

    @functools.partial(
        pl.kernel,
        out_type=jax.ShapeDtypeStruct((N,), jnp.int32),
        mesh=mesh,
        scratch_types=[
            pltpu.VMEM((W,), jnp.int32),
            pltpu.VMEM((W,), jnp.int32),
            pltpu.VMEM((BUF,), jnp.int32),
            pltpu.SemaphoreType.DMA,
            pltpu.SemaphoreType.DMA,
        ],
        compiler_params=pltpu.CompilerParams(needs_layout_passes=False),
    )
    def scatter_kernel(idx_hbm, out_hbm, win0_v, win1_v, out_v, sem0, sem1):
        wid = lax.axis_index("s") * 2 + lax.axis_index("c")
        base = wid * CHUNK
        rng = jnp.where(wid == NW - 1, LAST_CHUNK, CHUNK).astype(jnp.uint32)
        lane = lax.iota(jnp.int32, L)
        zeros = jnp.zeros((L,), jnp.int32)

        def wait(win_v, sem):
            pltpu.make_async_copy(idx_hbm.at[pl.ds(0, W)], win_v, sem).wait()

        def process(win_v, jbase):
            @plsc.parallel_loop(0, W // L, unroll=16)
            def _vec(i):
                vidx = win_v[pl.ds(i * L, L)]
                loc = vidx - base
                mask = loc.astype(jnp.uint32) < rng
                jv = (jbase + i * L) + lane
                plsc.store_scatter(out_v, [loc], jv, mask=mask)

        pltpu.async_copy(idx_hbm.at[pl.ds(0, W)], win0_v, sem0)

        @plsc.parallel_loop(0, BUF // L, unroll=8)
        def _zero(i):
            out_v[pl.ds(i * L, L)] = zeros

        @pl.loop(0, NPAIR)
        def _win(t):
            w0 = 2 * t
            pltpu.async_copy(idx_hbm.at[pl.ds((w0 + 1) * W, W)], win1_v, sem1)
            wait(win0_v, sem0)
            process(win0_v, w0 * W)
            pltpu.async_copy(idx_hbm.at[pl.ds((w0 + 2) * W, W)], win0_v, sem0)
            wait(win1_v, sem1)
            process(win1_v, (w0 + 1) * W)

        wait(win0_v, sem0)
        process(win0_v, 2 * NPAIR * W)

        @pl.when(wid == NW - 1)
        def _store_last():
            pltpu.sync_copy(out_v, out_hbm.at[pl.ds(base, LAST_CHUNK)])

        @pl.when(wid != NW - 1)
        def _store():
            pltpu.sync_copy(out_v.at[pl.ds(0, CHUNK)],
                            out_hbm.at[pl.ds(base, CHUNK)])

    return scatter_kernel


_sc_kernel = _make_sc_kernel()


def kernel(X, row_missing_idx):
    del X
    return _sc_kernel(row_missing_idx)


# speedup vs baseline: 40.1002x; 1.2049x over previous
import functools

import jax
import jax.numpy as jnp
from jax import lax
from jax.experimental import pallas as pl
from jax.experimental.pallas import tpu as pltpu
from jax.experimental.pallas import tpu_sc as plsc

N = 1_000_000
M = 500_000
NW = 32
NR = 16
RCHUNK = 62_496
RLAST = N - (NR - 1) * RCHUNK
RBUF = RLAST
MH = M // 2
W = 10_000
NWIN = MH // W
NPAIR = NWIN // 2
WM = 10_000
NMW = RBUF // WM
MTAIL = RBUF - NMW * WM
L = 16


def _make_sc_kernel():
    mesh = plsc.VectorSubcoreMesh(core_axis_name="c", subcore_axis_name="s",
                                  num_cores=2, num_subcores=16)

    @functools.partial(
        pl.kernel,
        out_type=jax.ShapeDtypeStruct((N,), jnp.int32),
        mesh=mesh,
        scratch_types=[
            pltpu.VMEM((W,), jnp.int32),
            pltpu.VMEM((W,), jnp.int32),
            pltpu.VMEM((RBUF,), jnp.int32),
            pltpu.VMEM_SHARED((NR // 2 * RBUF,), jnp.int32),
            pltpu.SemaphoreType.DMA,
            pltpu.SemaphoreType.DMA,
        ],
        compiler_params=pltpu.CompilerParams(needs_layout_passes=False),
    )
    def scatter_kernel(idx_hbm, out_hbm, win0_v, win1_v, out_v, stage_s,
                       sem0, sem1):
        c = lax.axis_index("c")
        s = lax.axis_index("s")
        pair = s >> 1
        rid = pair * 2 + c
        jhalf = s & 1
        base = rid * RCHUNK
        joff = jhalf * MH
        rng = jnp.where(rid == NR - 1, RLAST, RCHUNK).astype(jnp.uint32)
        lane = lax.iota(jnp.int32, L)
        zeros = jnp.zeros((L,), jnp.int32)

        def wait(win_v, sem):
            pltpu.make_async_copy(idx_hbm.at[pl.ds(0, W)], win_v, sem).wait()

        def process(win_v, jbase):
            @plsc.parallel_loop(0, W // L, unroll=16)
            def _vec(i):
                vidx = win_v[pl.ds(i * L, L)]
                loc = vidx - base
                mask = loc.astype(jnp.uint32) < rng
                jv = (jbase + i * L) + lane
                plsc.store_scatter(out_v, [loc], jv, mask=mask)

        pltpu.async_copy(idx_hbm.at[pl.ds(joff, W)], win0_v, sem0)

        @plsc.parallel_loop(0, RBUF // L, unroll=8)
        def _zero(i):
            out_v[pl.ds(i * L, L)] = zeros

        @pl.loop(0, NPAIR)
        def _win(t):
            w0 = 2 * t
            pltpu.async_copy(idx_hbm.at[pl.ds(joff + (w0 + 1) * W, W)],
                             win1_v, sem1)
            wait(win0_v, sem0)
            process(win0_v, joff + w0 * W)
            pltpu.async_copy(idx_hbm.at[pl.ds(joff + (w0 + 2) * W, W)],
                             win0_v, sem0)
            wait(win1_v, sem1)
            process(win1_v, joff + (w0 + 1) * W)

        wait(win0_v, sem0)
        process(win0_v, joff + 2 * NPAIR * W)

        plsc.subcore_barrier()

        slot = pl.multiple_of(pair * RBUF, 8)

        @pl.when(jhalf == 0)
        def _stage():
            pltpu.sync_copy(out_v, stage_s.at[pl.ds(slot, RBUF)])

        plsc.subcore_barrier()

        @pl.when(jhalf == 1)
        def _merge():
            def merge_window(off, nvec, buf):
                @plsc.parallel_loop(0, nvec, unroll=8)
                def _m(i):
                    a = out_v[pl.ds(off + i * L, L)]
                    b = buf[pl.ds(i * L, L)]
                    out_v[pl.ds(off + i * L, L)] = jnp.maximum(a, b)

            @pl.loop(0, NMW)
            def _mw(w):
                off = w * WM
                pltpu.sync_copy(
                    stage_s.at[pl.ds(pl.multiple_of(slot + off, 8), WM)],
                    win0_v)
                merge_window(off, WM // L, win0_v)

            pltpu.sync_copy(
                stage_s.at[pl.ds(pl.multiple_of(slot + NMW * WM, 8), MTAIL)],
                win1_v.at[pl.ds(0, MTAIL)])
            merge_window(NMW * WM, MTAIL // L, win1_v)

            @pl.when(rid == NR - 1)
            def _store_last():
                pltpu.sync_copy(out_v, out_hbm.at[pl.ds(base, RLAST)])

            @pl.when(rid != NR - 1)
            def _store():
                pltpu.sync_copy(out_v.at[pl.ds(0, RCHUNK)],
                                out_hbm.at[pl.ds(base, RCHUNK)])

    return scatter_kernel


_sc_kernel = _make_sc_kernel()


def kernel(X, row_missing_idx):
    del X
    return _sc_kernel(row_missing_idx)
